# baseline (speedup 1.0000x reference)
import functools

import jax
import jax.numpy as jnp
import numpy as np
from jax import lax
from jax.experimental import pallas as pl
from jax.experimental.pallas import tpu as pltpu
from jax.experimental.pallas import tpu_sc as plsc

VOCAB = 1000
BATCH = 16384
DIM_A = 32
DIM_B = 64
DIM_F = 128
MAX_NORM_A = float(np.sqrt(DIM_A))
MAX_NORM_B = float(np.sqrt(DIM_B))

_INFO = plsc.get_sparse_core_info()
_NC = _INFO.num_cores
_NS = _INFO.num_subcores
_NW = _NC * _NS
_BPW = BATCH // _NW
_CHUNK = 128
_NCHUNK = _BPW // _CHUNK


def _renorm_body(ta_ref, tb_ref, of_ref):
    a = ta_ref[...]
    na = jnp.sqrt(jnp.sum(a * a, axis=1, keepdims=True))
    sa = jnp.where(na > MAX_NORM_A, MAX_NORM_A / (na + 1e-7), 1.0)
    b = tb_ref[...]
    nb = jnp.sqrt(jnp.sum(b * b, axis=1, keepdims=True))
    sb = jnp.where(nb > MAX_NORM_B, MAX_NORM_B / (nb + 1e-7), 1.0)
    pad = jnp.zeros((VOCAB, DIM_F - DIM_A - DIM_B), jnp.float32)
    of_ref[...] = jnp.concatenate([a * sa, b * sb, pad], axis=1)


_renorm = pl.pallas_call(
    _renorm_body,
    out_shape=jax.ShapeDtypeStruct((VOCAB, DIM_F), jnp.float32),
)


@functools.partial(
    pl.kernel,
    mesh=plsc.VectorSubcoreMesh(core_axis_name="c", subcore_axis_name="s"),
    out_type=jax.ShapeDtypeStruct((BATCH, DIM_F), jnp.float32),
    scratch_types=[
        pltpu.VMEM((_BPW,), jnp.int32),
        pltpu.VMEM((_BPW, DIM_F), jnp.float32),
        pltpu.SemaphoreType.DMA,
    ],
    compiler_params=pltpu.CompilerParams(use_tc_tiling_on_sc=True),
)
def _gather(ids_hbm, tf_hbm, of_hbm, idx_v, rows_f, sem):
    wid = lax.axis_index("s") * _NC + lax.axis_index("c")
    base = wid * _BPW
    pltpu.sync_copy(ids_hbm.at[pl.ds(base, _BPW)], idx_v)
    copies = []
    for j in range(_NCHUNK):
        sl = pl.ds(j * _CHUNK, _CHUNK)
        copies.append(pltpu.async_copy(tf_hbm.at[idx_v.at[sl]], rows_f.at[sl], sem))
    for c in copies:
        c.wait()
    pltpu.sync_copy(rows_f, of_hbm.at[pl.ds(base, _BPW)])


@jax.jit
def kernel(ids, table_a, table_b):
    tf = _renorm(table_a, table_b)
    of = _gather(ids, tf)
    return of[:, :DIM_A], of[:, DIM_A:DIM_A + DIM_B]
